# baseline (speedup 1.0000x reference)
import functools

import jax
import jax.numpy as jnp
from jax import lax
from jax.experimental import pallas as pl
from jax.experimental.pallas import tpu as pltpu
from jax.experimental.pallas import tpu_sc as plsc

K = 8
NUM_CLASSES = 64
INPUT_DIM = 1 + NUM_CLASSES
B, L = 128, 512
LMAX = L - K
D = K * INPUT_DIM


ROWS_PER_STEP = 8


def _tc_body(pred_a, pred_b, dmat_a, dmat_b, ltgt_a, ltgt_b,
             msk_a, msk_b, ssel_ref, rrep_ref, dsel_ref, out_ref):
    g = pl.program_id(0)

    @pl.when(g == 0)
    def _init():
        out_ref[...] = jnp.zeros((4, D), jnp.float32)

    lane = jax.lax.broadcasted_iota(jnp.int32, (L, D), 1)
    laneoff = (lane - (lane // INPUT_DIM) * INPUT_DIM - 1
               ).astype(jnp.float32)
    eyek = (jax.lax.broadcasted_iota(jnp.int32, (K, K), 0) ==
            jax.lax.broadcasted_iota(jnp.int32, (K, K), 1)
            ).astype(jnp.bfloat16)
    for h, (pr, dr, lr, mr) in enumerate(
            ((pred_a, dmat_a, ltgt_a, msk_a),
             (pred_b, dmat_b, ltgt_b, msk_b))):
        for r in range(ROWS_PER_STEP):
            _tc_row(pr, dr, lr, mr, ssel_ref, rrep_ref,
                    dsel_ref, out_ref, laneoff, eyek, r)


def _tc_row(pred_ref, dmat_ref, ltgt_ref, msk_ref, ssel_ref, rrep_ref,
            dsel_ref, out_ref, laneoff, eyek, r):
    x = pred_ref[r]
    x_bf = x.astype(jnp.bfloat16)
    dmat = dmat_ref[r]
    ltgt = ltgt_ref[r]
    ones_m = msk_ref[r:r + 1, :]

    e_bf = jnp.exp(x).astype(jnp.bfloat16)
    sums = jax.lax.dot_general(
        ssel_ref[...], e_bf, (((1,), (1,)), ((), ())),
        preferred_element_type=jnp.float32)
    lse_k = jnp.sum(jnp.log(sums), axis=0, keepdims=True)
    lse_row = jnp.where(ones_m > 0.0, lse_k, 0.0)

    lblb = jax.lax.dot_general(
        ltgt.astype(jnp.bfloat16), rrep_ref[...], (((0,), (0,)), ((), ())),
        preferred_element_type=jnp.float32)
    sel = jnp.where(lblb == laneoff, x_bf, jnp.bfloat16(0))
    ones_all = jnp.zeros((1, L), jnp.bfloat16) + jnp.bfloat16(1)
    p1 = jax.lax.dot_general(
        ones_all, sel, (((1,), (0,)), ((), ())),
        preferred_element_type=jnp.float32)

    xd = jax.lax.dot_general(
        x_bf, dsel_ref[...], (((1,), (1,)), ((), ())),
        preferred_element_type=jnp.float32)
    dmat_t = jax.lax.dot_general(
        dmat.astype(jnp.bfloat16), eyek, (((0,), (0,)), ((), ())),
        preferred_element_type=jnp.float32)
    sqk = (xd - dmat_t) ** 2
    t1 = jax.lax.dot_general(
        ones_m.astype(jnp.bfloat16), sqk.astype(jnp.bfloat16),
        (((1,), (0,)), ((), ())),
        preferred_element_type=jnp.float32)

    out_ref[0:1, 0:K] += t1
    out_ref[1:2, 0:L] += lse_row
    out_ref[2:3, :] += p1
    out_ref[3:4, 0:L] += ones_m


def _build_consts():
    c = jnp.arange(D, dtype=jnp.int32)
    seg = c // INPUT_DIM
    off = c % INPUT_DIM
    t = jnp.arange(K, dtype=jnp.int32)
    ssel = ((seg[None, :] == t[:, None]) & (off[None, :] > 0)
            ).astype(jnp.float32)
    rrep = (seg[None, :] == t[:, None]).astype(jnp.float32)
    dsel = ((seg[None, :] == t[:, None]) & (off[None, :] == 0)
            ).astype(jnp.float32)
    return ssel, rrep, dsel


_VB = 16
_TSBUF = L + K


def _sc_body(ts_hbm, lbl_hbm, seq_hbm, dmat_hbm, ltgt_hbm, msk_hbm,
             ts_v, lbl_v, seq_v, dm_v, lt_v, mk_v):
    nc = 2
    wid = lax.axis_index("s") * nc + lax.axis_index("c")
    rpw = B // 32
    iota = lax.iota(jnp.int32, _VB)
    zf = jnp.zeros((_VB,), jnp.float32)
    zi = jnp.zeros((_VB,), jnp.int32)
    pltpu.sync_copy(seq_hbm, seq_v.at[pl.ds(0, B)])
    for j in range(rpw):
        row = wid * rpw + j
        ts_v[pl.ds(L - _VB + K, _VB)] = zf
        lbl_v[pl.ds(L - _VB + K, _VB)] = zi
        pltpu.sync_copy(ts_hbm.at[row], ts_v.at[pl.ds(0, L)])
        pltpu.sync_copy(lbl_hbm.at[row], lbl_v.at[pl.ds(0, L)])
        length = seq_v[pl.ds(row, _VB)][0] - K

        def body(k, carry):
            i0 = k * _VB
            valid = (i0 + iota) < length
            mk_v[pl.ds(i0, _VB)] = jnp.where(valid, 1.0, 0.0)
            for t in range(K):
                a = ts_v[pl.ds(i0 + t, _VB)]
                b2 = ts_v[pl.ds(i0 + t + 1, _VB)]
                lbl = lbl_v[pl.ds(i0 + t + 1, _VB)]
                dm_v[t, pl.ds(i0, _VB)] = jnp.where(valid, b2 - a, 0.0)
                lt_v[t, pl.ds(i0, _VB)] = jnp.where(
                    valid, lbl.astype(jnp.float32), -1000.0)
            return carry

        lax.fori_loop(0, L // _VB, body, 0)
        pltpu.sync_copy(dm_v, dmat_hbm.at[row])
        pltpu.sync_copy(lt_v, ltgt_hbm.at[row])
        pltpu.sync_copy(mk_v, msk_hbm.at[row])


@functools.partial(
    pl.kernel,
    mesh=plsc.VectorSubcoreMesh(core_axis_name="c", subcore_axis_name="s"),
    out_type=[
        jax.ShapeDtypeStruct((B, K, L), jnp.float32),
        jax.ShapeDtypeStruct((B, K, L), jnp.float32),
        jax.ShapeDtypeStruct((B, L), jnp.float32),
    ],
    scratch_types=[
        pltpu.VMEM((_TSBUF,), jnp.float32),
        pltpu.VMEM((_TSBUF,), jnp.int32),
        pltpu.VMEM((B + _VB,), jnp.int32),
        pltpu.VMEM((K, L), jnp.float32),
        pltpu.VMEM((K, L), jnp.float32),
        pltpu.VMEM((L,), jnp.float32),
    ],
)
def _build_aux_sc(ts_hbm, lbl_hbm, seq_hbm, dmat_hbm, ltgt_hbm, msk_hbm,
                  ts_v, lbl_v, seq_v, dm_v, lt_v, mk_v):
    _sc_body(ts_hbm, lbl_hbm, seq_hbm, dmat_hbm, ltgt_hbm, msk_hbm,
             ts_v, lbl_v, seq_v, dm_v, lt_v, mk_v)


def kernel(timestamps, labels, seq_lens, predictions):
    ssel, rrep, dsel = _build_consts()
    dmat, ltgt, msk = _build_aux_sc(timestamps, labels.astype(jnp.int32),
                                    seq_lens.astype(jnp.int32))
    rps = ROWS_PER_STEP
    sums = pl.pallas_call(
        _tc_body,
        grid=(B // (2 * rps),),
        in_specs=[
            pl.BlockSpec((rps, L, D), lambda b: (2 * b, 0, 0)),
            pl.BlockSpec((rps, L, D), lambda b: (2 * b + 1, 0, 0)),
            pl.BlockSpec((rps, K, L), lambda b: (2 * b, 0, 0)),
            pl.BlockSpec((rps, K, L), lambda b: (2 * b + 1, 0, 0)),
            pl.BlockSpec((rps, K, L), lambda b: (2 * b, 0, 0)),
            pl.BlockSpec((rps, K, L), lambda b: (2 * b + 1, 0, 0)),
            pl.BlockSpec((rps, L), lambda b: (2 * b, 0)),
            pl.BlockSpec((rps, L), lambda b: (2 * b + 1, 0)),
            pl.BlockSpec((K, D), lambda b: (0, 0)),
            pl.BlockSpec((K, D), lambda b: (0, 0)),
            pl.BlockSpec((K, D), lambda b: (0, 0)),
        ],
        out_specs=pl.BlockSpec((4, D), lambda b: (0, 0)),
        out_shape=jax.ShapeDtypeStruct((4, D), jnp.float32),
    )(predictions, predictions, dmat, dmat, ltgt, ltgt, msk, msk,
      ssel.astype(jnp.bfloat16), rrep.astype(jnp.bfloat16),
      dsel.astype(jnp.bfloat16))
    ts_num = jnp.sum(sums[0])
    lbl_num = jnp.sum(sums[1]) - jnp.sum(sums[2])
    n = jnp.sum(sums[3])
    return jnp.stack([ts_num, lbl_num]) / (n * jnp.float32(K))


# speedup vs baseline: 1.0008x; 1.0008x over previous
import functools

import jax
import jax.numpy as jnp
from jax import lax
from jax.experimental import pallas as pl
from jax.experimental.pallas import tpu as pltpu
from jax.experimental.pallas import tpu_sc as plsc

K = 8
NUM_CLASSES = 64
INPUT_DIM = 1 + NUM_CLASSES
B, L = 128, 512
LMAX = L - K
D = K * INPUT_DIM


ROWS_PER_STEP = 8


def _tc_body(pred_a, dmat_a, ltgt_a, msk_a,
             ssel_ref, rrep_ref, dsel_ref, out_ref):
    g = pl.program_id(0)

    @pl.when(g == 0)
    def _init():
        out_ref[...] = jnp.zeros((4, D), jnp.float32)

    lane = jax.lax.broadcasted_iota(jnp.int32, (L, D), 1)
    laneoff = (lane - (lane // INPUT_DIM) * INPUT_DIM - 1
               ).astype(jnp.float32)
    eyek = (jax.lax.broadcasted_iota(jnp.int32, (K, K), 0) ==
            jax.lax.broadcasted_iota(jnp.int32, (K, K), 1)
            ).astype(jnp.bfloat16)
    for r in range(ROWS_PER_STEP):
        _tc_row(pred_a, dmat_a, ltgt_a, msk_a, ssel_ref, rrep_ref,
                dsel_ref, out_ref, laneoff, eyek, r)


def _tc_row(pred_ref, dmat_ref, ltgt_ref, msk_ref, ssel_ref, rrep_ref,
            dsel_ref, out_ref, laneoff, eyek, r):
    x = pred_ref[r]
    x_bf = x.astype(jnp.bfloat16)
    dmat = dmat_ref[r]
    ltgt = ltgt_ref[r]
    ones_m = msk_ref[r:r + 1, :]

    e_bf = jnp.exp(x).astype(jnp.bfloat16)
    sums = jax.lax.dot_general(
        ssel_ref[...], e_bf, (((1,), (1,)), ((), ())),
        preferred_element_type=jnp.float32)
    lse_k = jnp.sum(jnp.log(sums), axis=0, keepdims=True)
    lse_row = jnp.where(ones_m > 0.0, lse_k, 0.0)

    lblb = jax.lax.dot_general(
        ltgt.astype(jnp.bfloat16), rrep_ref[...], (((0,), (0,)), ((), ())),
        preferred_element_type=jnp.float32)
    sel = jnp.where(lblb == laneoff, x_bf, jnp.bfloat16(0))
    ones_all = jnp.zeros((1, L), jnp.bfloat16) + jnp.bfloat16(1)
    p1 = jax.lax.dot_general(
        ones_all, sel, (((1,), (0,)), ((), ())),
        preferred_element_type=jnp.float32)

    xd = jax.lax.dot_general(
        x_bf, dsel_ref[...], (((1,), (1,)), ((), ())),
        preferred_element_type=jnp.float32)
    dmat_t = jax.lax.dot_general(
        dmat.astype(jnp.bfloat16), eyek, (((0,), (0,)), ((), ())),
        preferred_element_type=jnp.float32)
    sqk = (xd - dmat_t) ** 2
    t1 = jax.lax.dot_general(
        ones_m.astype(jnp.bfloat16), sqk.astype(jnp.bfloat16),
        (((1,), (0,)), ((), ())),
        preferred_element_type=jnp.float32)

    out_ref[0:1, 0:K] += t1
    out_ref[1:2, 0:L] += lse_row
    out_ref[2:3, :] += p1
    out_ref[3:4, 0:L] += ones_m


def _build_consts():
    c = jnp.arange(D, dtype=jnp.int32)
    seg = c // INPUT_DIM
    off = c % INPUT_DIM
    t = jnp.arange(K, dtype=jnp.int32)
    ssel = ((seg[None, :] == t[:, None]) & (off[None, :] > 0)
            ).astype(jnp.float32)
    rrep = (seg[None, :] == t[:, None]).astype(jnp.float32)
    dsel = ((seg[None, :] == t[:, None]) & (off[None, :] == 0)
            ).astype(jnp.float32)
    return ssel, rrep, dsel


_VB = 16
_TSBUF = L + K


def _sc_body(ts_hbm, lbl_hbm, seq_hbm, dmat_hbm, ltgt_hbm, msk_hbm,
             ts_v, lbl_v, seq_v, dm_v, lt_v, mk_v):
    nc = 2
    wid = lax.axis_index("s") * nc + lax.axis_index("c")
    rpw = B // 32
    iota = lax.iota(jnp.int32, _VB)
    zf = jnp.zeros((_VB,), jnp.float32)
    zi = jnp.zeros((_VB,), jnp.int32)
    pltpu.sync_copy(seq_hbm, seq_v.at[pl.ds(0, B)])
    for j in range(rpw):
        row = wid * rpw + j
        ts_v[pl.ds(L - _VB + K, _VB)] = zf
        lbl_v[pl.ds(L - _VB + K, _VB)] = zi
        pltpu.sync_copy(ts_hbm.at[row], ts_v.at[pl.ds(0, L)])
        pltpu.sync_copy(lbl_hbm.at[row], lbl_v.at[pl.ds(0, L)])
        length = seq_v[pl.ds(row, _VB)][0] - K

        def body(k, carry):
            i0 = k * _VB
            valid = (i0 + iota) < length
            mk_v[pl.ds(i0, _VB)] = jnp.where(valid, 1.0, 0.0)
            for t in range(K):
                a = ts_v[pl.ds(i0 + t, _VB)]
                b2 = ts_v[pl.ds(i0 + t + 1, _VB)]
                lbl = lbl_v[pl.ds(i0 + t + 1, _VB)]
                dm_v[t, pl.ds(i0, _VB)] = jnp.where(valid, b2 - a, 0.0)
                lt_v[t, pl.ds(i0, _VB)] = jnp.where(
                    valid, lbl.astype(jnp.float32), -1000.0)
            return carry

        lax.fori_loop(0, L // _VB, body, 0)
        pltpu.sync_copy(dm_v, dmat_hbm.at[row])
        pltpu.sync_copy(lt_v, ltgt_hbm.at[row])
        pltpu.sync_copy(mk_v, msk_hbm.at[row])


@functools.partial(
    pl.kernel,
    mesh=plsc.VectorSubcoreMesh(core_axis_name="c", subcore_axis_name="s"),
    out_type=[
        jax.ShapeDtypeStruct((B, K, L), jnp.float32),
        jax.ShapeDtypeStruct((B, K, L), jnp.float32),
        jax.ShapeDtypeStruct((B, L), jnp.float32),
    ],
    scratch_types=[
        pltpu.VMEM((_TSBUF,), jnp.float32),
        pltpu.VMEM((_TSBUF,), jnp.int32),
        pltpu.VMEM((B + _VB,), jnp.int32),
        pltpu.VMEM((K, L), jnp.float32),
        pltpu.VMEM((K, L), jnp.float32),
        pltpu.VMEM((L,), jnp.float32),
    ],
)
def _build_aux_sc(ts_hbm, lbl_hbm, seq_hbm, dmat_hbm, ltgt_hbm, msk_hbm,
                  ts_v, lbl_v, seq_v, dm_v, lt_v, mk_v):
    _sc_body(ts_hbm, lbl_hbm, seq_hbm, dmat_hbm, ltgt_hbm, msk_hbm,
             ts_v, lbl_v, seq_v, dm_v, lt_v, mk_v)


def kernel(timestamps, labels, seq_lens, predictions):
    ssel, rrep, dsel = _build_consts()
    dmat, ltgt, msk = _build_aux_sc(timestamps, labels.astype(jnp.int32),
                                    seq_lens.astype(jnp.int32))
    rps = ROWS_PER_STEP
    sums = pl.pallas_call(
        _tc_body,
        grid=(B // rps,),
        in_specs=[
            pl.BlockSpec((rps, L, D), lambda b: (b, 0, 0)),
            pl.BlockSpec((rps, K, L), lambda b: (b, 0, 0)),
            pl.BlockSpec((rps, K, L), lambda b: (b, 0, 0)),
            pl.BlockSpec((rps, L), lambda b: (b, 0)),
            pl.BlockSpec((K, D), lambda b: (0, 0)),
            pl.BlockSpec((K, D), lambda b: (0, 0)),
            pl.BlockSpec((K, D), lambda b: (0, 0)),
        ],
        out_specs=pl.BlockSpec((4, D), lambda b: (0, 0)),
        out_shape=jax.ShapeDtypeStruct((4, D), jnp.float32),
    )(predictions, dmat, ltgt, msk,
      ssel.astype(jnp.bfloat16), rrep.astype(jnp.bfloat16),
      dsel.astype(jnp.bfloat16))
    ts_num = jnp.sum(sums[0])
    lbl_num = jnp.sum(sums[1]) - jnp.sum(sums[2])
    n = jnp.sum(sums[3])
    return jnp.stack([ts_num, lbl_num]) / (n * jnp.float32(K))


# speedup vs baseline: 1.0184x; 1.0176x over previous
import functools

import jax
import jax.numpy as jnp
from jax import lax
from jax.experimental import pallas as pl
from jax.experimental.pallas import tpu as pltpu
from jax.experimental.pallas import tpu_sc as plsc

K = 8
NUM_CLASSES = 64
INPUT_DIM = 1 + NUM_CLASSES
B, L = 128, 512
LMAX = L - K
D = K * INPUT_DIM


ROWS_PER_STEP = 8


def _tc_body(pred_a, dmat_a, ltgt_a, msk_a,
             ssel_ref, rrep_ref, dsel_ref, out_ref):
    g = pl.program_id(0)

    @pl.when(g == 0)
    def _init():
        out_ref[...] = jnp.zeros((4, D), jnp.float32)

    lane = jax.lax.broadcasted_iota(jnp.int32, (L, D), 1)
    laneoff = (lane - (lane // INPUT_DIM) * INPUT_DIM - 1
               ).astype(jnp.float32)
    for r in range(ROWS_PER_STEP):
        _tc_row(pred_a, dmat_a, ltgt_a, msk_a, ssel_ref, rrep_ref,
                dsel_ref, out_ref, laneoff, r)


def _tc_row(pred_ref, dmat_ref, ltgt_ref, msk_ref, ssel_ref, rrep_ref,
            dsel_ref, out_ref, laneoff, r):
    x = pred_ref[r]
    dmat = dmat_ref[r]
    ltgt = ltgt_ref[r]
    ones_m = msk_ref[r:r + 1, :]

    e = jnp.exp(x)
    sums = jax.lax.dot_general(
        ssel_ref[...], e, (((1,), (1,)), ((), ())),
        preferred_element_type=jnp.float32)
    lse_k = jnp.sum(jnp.log(sums), axis=0, keepdims=True)
    lse_row = jnp.where(ones_m > 0.0, lse_k, 0.0)

    lblb = jax.lax.dot_general(
        ltgt, rrep_ref[...], (((0,), (0,)), ((), ())),
        preferred_element_type=jnp.float32)
    sel = jnp.where(lblb == laneoff, x, 0.0)
    ones_all = jnp.zeros((1, L), jnp.float32) + 1.0
    p1 = jax.lax.dot_general(
        ones_all, sel, (((1,), (0,)), ((), ())),
        preferred_element_type=jnp.float32)

    deltal = jax.lax.dot_general(
        dmat, dsel_ref[...], (((0,), (0,)), ((), ())),
        preferred_element_type=jnp.float32)
    sq = (x - deltal) ** 2
    t1 = jax.lax.dot_general(
        ones_m, sq, (((1,), (0,)), ((), ())),
        preferred_element_type=jnp.float32)
    dany = jnp.sum(dsel_ref[...], axis=0, keepdims=True)

    out_ref[0:1, :] += t1 * dany
    out_ref[1:2, 0:L] += lse_row
    out_ref[2:3, :] += p1
    out_ref[3:4, 0:L] += ones_m


def _build_consts():
    c = jnp.arange(D, dtype=jnp.int32)
    seg = c // INPUT_DIM
    off = c % INPUT_DIM
    t = jnp.arange(K, dtype=jnp.int32)
    ssel = ((seg[None, :] == t[:, None]) & (off[None, :] > 0)
            ).astype(jnp.float32)
    rrep = (seg[None, :] == t[:, None]).astype(jnp.float32)
    dsel = ((seg[None, :] == t[:, None]) & (off[None, :] == 0)
            ).astype(jnp.float32)
    return ssel, rrep, dsel


_VB = 16
_TSBUF = L + K


def _sc_body(ts_hbm, lbl_hbm, seq_hbm, dmat_hbm, ltgt_hbm, msk_hbm,
             ts_v, lbl_v, seq_v, dm_v, lt_v, mk_v):
    nc = 2
    wid = lax.axis_index("s") * nc + lax.axis_index("c")
    rpw = B // 32
    iota = lax.iota(jnp.int32, _VB)
    zf = jnp.zeros((_VB,), jnp.float32)
    zi = jnp.zeros((_VB,), jnp.int32)
    pltpu.sync_copy(seq_hbm, seq_v.at[pl.ds(0, B)])
    for j in range(rpw):
        row = wid * rpw + j
        ts_v[pl.ds(L - _VB + K, _VB)] = zf
        lbl_v[pl.ds(L - _VB + K, _VB)] = zi
        pltpu.sync_copy(ts_hbm.at[row], ts_v.at[pl.ds(0, L)])
        pltpu.sync_copy(lbl_hbm.at[row], lbl_v.at[pl.ds(0, L)])
        length = seq_v[pl.ds(row, _VB)][0] - K

        def body(k, carry):
            i0 = k * _VB
            valid = (i0 + iota) < length
            mk_v[pl.ds(i0, _VB)] = jnp.where(valid, 1.0, 0.0)
            for t in range(K):
                a = ts_v[pl.ds(i0 + t, _VB)]
                b2 = ts_v[pl.ds(i0 + t + 1, _VB)]
                lbl = lbl_v[pl.ds(i0 + t + 1, _VB)]
                dm_v[t, pl.ds(i0, _VB)] = jnp.where(valid, b2 - a, 0.0)
                lt_v[t, pl.ds(i0, _VB)] = jnp.where(
                    valid, lbl.astype(jnp.float32), -1000.0)
            return carry

        lax.fori_loop(0, L // _VB, body, 0)
        pltpu.sync_copy(dm_v, dmat_hbm.at[row])
        pltpu.sync_copy(lt_v, ltgt_hbm.at[row])
        pltpu.sync_copy(mk_v, msk_hbm.at[row])


@functools.partial(
    pl.kernel,
    mesh=plsc.VectorSubcoreMesh(core_axis_name="c", subcore_axis_name="s"),
    out_type=[
        jax.ShapeDtypeStruct((B, K, L), jnp.float32),
        jax.ShapeDtypeStruct((B, K, L), jnp.float32),
        jax.ShapeDtypeStruct((B, L), jnp.float32),
    ],
    scratch_types=[
        pltpu.VMEM((_TSBUF,), jnp.float32),
        pltpu.VMEM((_TSBUF,), jnp.int32),
        pltpu.VMEM((B + _VB,), jnp.int32),
        pltpu.VMEM((K, L), jnp.float32),
        pltpu.VMEM((K, L), jnp.float32),
        pltpu.VMEM((L,), jnp.float32),
    ],
)
def _build_aux_sc(ts_hbm, lbl_hbm, seq_hbm, dmat_hbm, ltgt_hbm, msk_hbm,
                  ts_v, lbl_v, seq_v, dm_v, lt_v, mk_v):
    _sc_body(ts_hbm, lbl_hbm, seq_hbm, dmat_hbm, ltgt_hbm, msk_hbm,
             ts_v, lbl_v, seq_v, dm_v, lt_v, mk_v)


def kernel(timestamps, labels, seq_lens, predictions):
    ssel, rrep, dsel = _build_consts()
    dmat, ltgt, msk = _build_aux_sc(timestamps, labels.astype(jnp.int32),
                                    seq_lens.astype(jnp.int32))
    rps = ROWS_PER_STEP
    sums = pl.pallas_call(
        _tc_body,
        grid=(B // rps,),
        in_specs=[
            pl.BlockSpec((rps, L, D), lambda b: (b, 0, 0)),
            pl.BlockSpec((rps, K, L), lambda b: (b, 0, 0)),
            pl.BlockSpec((rps, K, L), lambda b: (b, 0, 0)),
            pl.BlockSpec((rps, L), lambda b: (b, 0)),
            pl.BlockSpec((K, D), lambda b: (0, 0)),
            pl.BlockSpec((K, D), lambda b: (0, 0)),
            pl.BlockSpec((K, D), lambda b: (0, 0)),
        ],
        out_specs=pl.BlockSpec((4, D), lambda b: (0, 0)),
        out_shape=jax.ShapeDtypeStruct((4, D), jnp.float32),
    )(predictions, dmat, ltgt, msk, ssel, rrep, dsel)
    ts_num = jnp.sum(sums[0])
    lbl_num = jnp.sum(sums[1]) - jnp.sum(sums[2])
    n = jnp.sum(sums[3])
    return jnp.stack([ts_num, lbl_num]) / (n * jnp.float32(K))


# speedup vs baseline: 1.0273x; 1.0087x over previous
import functools

import jax
import jax.numpy as jnp
from jax import lax
from jax.experimental import pallas as pl
from jax.experimental.pallas import tpu as pltpu
from jax.experimental.pallas import tpu_sc as plsc

K = 8
NUM_CLASSES = 64
INPUT_DIM = 1 + NUM_CLASSES
B, L = 128, 512
LMAX = L - K
D = K * INPUT_DIM


ROWS_PER_STEP = 8


def _tc_body(seq_ref, pred_ref, dmat_ref, ltgt_ref, ssel_ref, rrep_ref,
             dsel_ref, out_ref):
    g = pl.program_id(0)

    @pl.when(g == 0)
    def _init():
        out_ref[...] = jnp.zeros((4, D), jnp.float32)

    for r in range(ROWS_PER_STEP):
        _tc_row(seq_ref, pred_ref, dmat_ref, ltgt_ref, ssel_ref, rrep_ref,
                dsel_ref, out_ref, g * ROWS_PER_STEP + r, r)


def _tc_row(seq_ref, pred_ref, dmat_ref, ltgt_ref, ssel_ref, rrep_ref,
            dsel_ref, out_ref, b, r):
    length = jnp.maximum(seq_ref[b] - K, 0)
    x = pred_ref[r]
    dmat = dmat_ref[r]
    ltgt = ltgt_ref[r]

    ones_m = (jax.lax.broadcasted_iota(jnp.int32, (1, L), 1)
              < length).astype(jnp.float32)

    e = jnp.exp(x)
    sums = jax.lax.dot_general(
        ssel_ref[...], e, (((1,), (1,)), ((), ())),
        preferred_element_type=jnp.float32)
    lse = jnp.log(sums)
    lmask = (jax.lax.broadcasted_iota(jnp.int32, (K, L), 1)
             < length)
    lse_row = jnp.sum(jnp.where(lmask, lse, 0.0), axis=0,
                      keepdims=True)

    ltl = jax.lax.dot_general(
        ltgt, rrep_ref[...], (((0,), (0,)), ((), ())),
        preferred_element_type=jnp.float32)
    lane = jax.lax.broadcasted_iota(jnp.int32, (L, D), 1
                                    ).astype(jnp.float32)
    sel = jnp.where(ltl == lane, x, 0.0)
    ones_all = jnp.zeros((1, L), jnp.float32) + 1.0
    p1 = jax.lax.dot_general(
        ones_all, sel, (((1,), (0,)), ((), ())),
        preferred_element_type=jnp.float32)

    deltal = jax.lax.dot_general(
        dmat, dsel_ref[...], (((0,), (0,)), ((), ())),
        preferred_element_type=jnp.float32)
    sq = (x - deltal) ** 2
    t1 = jax.lax.dot_general(
        ones_m, sq, (((1,), (0,)), ((), ())),
        preferred_element_type=jnp.float32)
    dany = jnp.sum(dsel_ref[...], axis=0, keepdims=True)

    out_ref[0:1, :] += t1 * dany
    out_ref[1:2, 0:L] += lse_row
    out_ref[2:3, :] += p1
    out_ref[3:4, 0:L] += ones_m


def _build_consts():
    c = jnp.arange(D, dtype=jnp.int32)
    seg = c // INPUT_DIM
    off = c % INPUT_DIM
    t = jnp.arange(K, dtype=jnp.int32)
    ssel = ((seg[None, :] == t[:, None]) & (off[None, :] > 0)
            ).astype(jnp.float32)
    rrep = (seg[None, :] == t[:, None]).astype(jnp.float32)
    dsel = ((seg[None, :] == t[:, None]) & (off[None, :] == 0)
            ).astype(jnp.float32)
    return ssel, rrep, dsel


_VB = 16
_TSBUF = L + K


def _sc_body(ts_hbm, lbl_hbm, seq_hbm, dmat_hbm, ltgt_hbm,
             ts_v, lbl_v, seq_v, dm_v, lt_v):
    nc = 2
    wid = lax.axis_index("s") * nc + lax.axis_index("c")
    rpw = B // 32
    iota = lax.iota(jnp.int32, _VB)
    zf = jnp.zeros((_VB,), jnp.float32)
    zi = jnp.zeros((_VB,), jnp.int32)
    pltpu.sync_copy(seq_hbm, seq_v.at[pl.ds(0, B)])
    for j in range(rpw):
        row = wid * rpw + j
        ts_v[pl.ds(L - _VB + K, _VB)] = zf
        lbl_v[pl.ds(L - _VB + K, _VB)] = zi
        pltpu.sync_copy(ts_hbm.at[row], ts_v.at[pl.ds(0, L)])
        pltpu.sync_copy(lbl_hbm.at[row], lbl_v.at[pl.ds(0, L)])
        length = seq_v[pl.ds(row, _VB)][0] - K

        def body(k, carry):
            i0 = k * _VB
            valid = (i0 + iota) < length
            for t in range(K):
                a = ts_v[pl.ds(i0 + t, _VB)]
                b2 = ts_v[pl.ds(i0 + t + 1, _VB)]
                lbl = lbl_v[pl.ds(i0 + t + 1, _VB)]
                dm_v[t, pl.ds(i0, _VB)] = jnp.where(valid, b2 - a, 0.0)
                lt = (lbl + (t * INPUT_DIM + 1)).astype(jnp.float32)
                lt_v[t, pl.ds(i0, _VB)] = jnp.where(valid, lt, -1000.0)
            return carry

        lax.fori_loop(0, L // _VB, body, 0)
        pltpu.sync_copy(dm_v, dmat_hbm.at[row])
        pltpu.sync_copy(lt_v, ltgt_hbm.at[row])


@functools.partial(
    pl.kernel,
    mesh=plsc.VectorSubcoreMesh(core_axis_name="c", subcore_axis_name="s"),
    out_type=[
        jax.ShapeDtypeStruct((B, K, L), jnp.float32),
        jax.ShapeDtypeStruct((B, K, L), jnp.float32),
    ],
    scratch_types=[
        pltpu.VMEM((_TSBUF,), jnp.float32),
        pltpu.VMEM((_TSBUF,), jnp.int32),
        pltpu.VMEM((B + _VB,), jnp.int32),
        pltpu.VMEM((K, L), jnp.float32),
        pltpu.VMEM((K, L), jnp.float32),
    ],
)
def _build_aux_sc(ts_hbm, lbl_hbm, seq_hbm, dmat_hbm, ltgt_hbm,
                  ts_v, lbl_v, seq_v, dm_v, lt_v):
    _sc_body(ts_hbm, lbl_hbm, seq_hbm, dmat_hbm, ltgt_hbm,
             ts_v, lbl_v, seq_v, dm_v, lt_v)


def kernel(timestamps, labels, seq_lens, predictions):
    ssel, rrep, dsel = _build_consts()
    dmat, ltgt = _build_aux_sc(timestamps, labels.astype(jnp.int32),
                               seq_lens.astype(jnp.int32))
    sums = pl.pallas_call(
        _tc_body,
        grid=(B // ROWS_PER_STEP,),
        in_specs=[
            pl.BlockSpec(memory_space=pltpu.SMEM),
            pl.BlockSpec((ROWS_PER_STEP, L, D), lambda b: (b, 0, 0)),
            pl.BlockSpec((ROWS_PER_STEP, K, L), lambda b: (b, 0, 0)),
            pl.BlockSpec((ROWS_PER_STEP, K, L), lambda b: (b, 0, 0)),
            pl.BlockSpec((K, D), lambda b: (0, 0)),
            pl.BlockSpec((K, D), lambda b: (0, 0)),
            pl.BlockSpec((K, D), lambda b: (0, 0)),
        ],
        out_specs=pl.BlockSpec((4, D), lambda b: (0, 0)),
        out_shape=jax.ShapeDtypeStruct((4, D), jnp.float32),
    )(seq_lens.astype(jnp.int32), predictions, dmat, ltgt, ssel, rrep, dsel)
    ts_num = jnp.sum(sums[0])
    lbl_num = jnp.sum(sums[1]) - jnp.sum(sums[2])
    n = jnp.sum(sums[3])
    return jnp.stack([ts_num, lbl_num]) / (n * jnp.float32(K))
